# Initial kernel scaffold; baseline (speedup 1.0000x reference)
import jax
import jax.numpy as jnp
from jax.experimental import pallas as pl


def kernel(hidden, time_emb, edge_index, edge_type, edge_emb, Wq, bq, Wk, bk, Wv, bv, We, be, W1, b1, W2, b2, gamma, beta):
    raise NotImplementedError("write your pallas kernel here")


# speedup vs baseline: 1.5575x; 1.5575x over previous
import functools
import math

import jax
import jax.numpy as jnp
from jax import lax
from jax.experimental import pallas as pl
from jax.experimental.pallas import tpu as pltpu
from jax.experimental.pallas import tpu_sc as plsc

N = 10000
D = 128
E = 160000
T = 16

NC = 2
NS = 16
NW = NC * NS

C = 128
EP = ((E + NW - 1) // NW + C - 1) // C * C
E2 = EP * NW
NCH = EP // C
NR = 632
NP = NS * NR
SPT = 8
SR = NS * SPT

SCL = 1.0 / math.sqrt(D)
RB = 2000
NB = N // RB

_HI = jax.lax.Precision.HIGHEST


def _tc_proj_body(h_ref, t_ref, wq_ref, bq_ref, wk_ref, wv_ref,
                  q_ref, k_ref, v_ref):
    h = h_ref[...]
    ht = h + t_ref[...]
    q_ref[...] = jnp.dot(ht, wq_ref[...], preferred_element_type=jnp.float32,
                         precision=_HI) + bq_ref[...]
    k_ref[...] = jnp.dot(ht, wk_ref[...], preferred_element_type=jnp.float32,
                         precision=_HI)
    v_ref[...] = jnp.dot(h, wv_ref[...], preferred_element_type=jnp.float32,
                         precision=_HI)


def _tc_finish_body(h_ref, t_ref, a0_ref, a1_ref, sm_ref, w1a_ref,
                    w1b_ref, w1c_ref, b1_ref, w2_ref, b2_ref, g_ref, bt_ref,
                    o_ref):
    h = h_ref[...]
    a = a0_ref[...] + a1_ref[...]
    sm = sm_ref[...]
    agg = jnp.where(sm > 0, a / jnp.maximum(sm, 1e-30), 0.0)
    u = (jnp.dot(h, w1a_ref[...], preferred_element_type=jnp.float32,
                 precision=_HI)
         + jnp.dot(agg, w1b_ref[...], preferred_element_type=jnp.float32,
                   precision=_HI)
         + jnp.dot(t_ref[...], w1c_ref[...],
                   preferred_element_type=jnp.float32, precision=_HI)
         + b1_ref[...])
    u = u / (1.0 + jnp.exp(-u))
    y = jnp.dot(u, w2_ref[...], preferred_element_type=jnp.float32,
                precision=_HI) + b2_ref[...] + h
    mean = jnp.mean(y, axis=1, keepdims=True)
    yc = y - mean
    var = jnp.mean(yc * yc, axis=1, keepdims=True)
    o_ref[...] = yc * jax.lax.rsqrt(var + 1e-5) * g_ref[...] + bt_ref[...]


def _sc_logits_body(qn, kn, kt2, elog, tgt, src, typ, logit_o, tmax_o,
                    tgt_v, src_v, typ_v, q_v, k_v, kt_v, el_v, lg_v, mx_v,
                    sem):
    cid = lax.axis_index("c")
    sid = lax.axis_index("s")
    wid = sid * NC + cid
    base = wid * EP
    lanes = lax.iota(jnp.int32, 16)

    pltpu.sync_copy(kt2, kt_v)
    pltpu.sync_copy(elog, el_v)

    def chunk(ch, mx):
        e0 = base + ch * C
        pltpu.sync_copy(tgt.at[pl.ds(e0, C)], tgt_v)
        pltpu.sync_copy(src.at[pl.ds(e0, C)], src_v)
        pltpu.sync_copy(typ.at[pl.ds(e0, C)], typ_v)
        pltpu.async_copy(qn.at[tgt_v], q_v, sem).wait()
        pltpu.async_copy(kn.at[src_v], k_v, sem).wait()
        for g in range(C // 16):
            e16 = lanes + (g * 16)
            ty16 = typ_v[pl.ds(g * 16, 16)]

            def dstep(dd, acc):
                dsp = jnp.full((16,), dd, jnp.int32)
                q16 = plsc.load_gather(q_v, [e16, dsp])
                k16 = plsc.load_gather(k_v, [e16, dsp])
                kt16 = plsc.load_gather(kt_v, [ty16, dsp])
                return acc + q16 * (k16 + kt16)

            acc = lax.fori_loop(0, D, dstep, jnp.zeros((16,), jnp.float32))
            lg = acc * SCL + plsc.load_gather(el_v, [ty16])
            eg = e0 + g * 16 + lanes
            lg = jnp.where(eg < E, lg, jnp.full((16,), -3e38, jnp.float32))
            lg_v[pl.ds(g * 16, 16)] = lg
            mx = jnp.maximum(mx, lg)
        pltpu.sync_copy(lg_v, logit_o.at[pl.ds(e0, C)])
        return mx

    mx = lax.fori_loop(0, NCH, chunk, jnp.full((16,), -3e38, jnp.float32))
    for g in range(8):
        mx_v[pl.ds(g * 16, 16)] = mx
    pltpu.sync_copy(mx_v, tmax_o.at[pl.ds(wid * 128, 128)])


def _sc_agg_body(vn, vt, tgt, src, typ, logit, gvec, z128, agg_o, sum_o,
                 tgt_v, src_v, typ_v, v_v, sr_v, lg_v, vt_v, gx_v,
                 agg_s, sum_s, sem):
    cid = lax.axis_index("c")
    sid = lax.axis_index("s")
    wid = sid * NC + cid
    base = wid * EP
    row0 = sid * NR
    lanes = lax.iota(jnp.int32, 16)
    zero16 = jnp.zeros((16,), jnp.float32)

    pltpu.sync_copy(gvec, gx_v)
    pltpu.sync_copy(vt, vt_v)
    pltpu.sync_copy(z128.at[pl.ds(0, C)], sr_v)
    pltpu.sync_copy(z128, agg_s.at[pl.ds(row0, NR)])
    pltpu.sync_copy(z128.at[pl.ds(0, SPT)], sum_s.at[pl.ds(sid * SPT, SPT)])
    plsc.subcore_barrier()
    gx = gx_v[...]

    def chunk(ch, carry):
        e0 = base + ch * C
        pltpu.sync_copy(tgt.at[pl.ds(e0, C)], tgt_v)
        pltpu.sync_copy(src.at[pl.ds(e0, C)], src_v)
        pltpu.sync_copy(typ.at[pl.ds(e0, C)], typ_v)
        pltpu.sync_copy(logit.at[pl.ds(e0, C)], lg_v)
        pltpu.async_copy(vn.at[src_v], v_v, sem).wait()
        for g in range(C // 16):
            e16 = lanes + (g * 16)
            ty16 = typ_v[pl.ds(g * 16, 16)]
            t16 = tgt_v[pl.ds(g * 16, 16)]
            ex16 = jnp.exp(lg_v[pl.ds(g * 16, 16)] - gx)
            col16 = jnp.bitwise_and(t16, 127)
            plsc.store_scatter(sr_v, [e16, col16], ex16)

            def dstep(dd, c2):
                dsp = jnp.full((16,), dd, jnp.int32)
                va = plsc.load_gather(v_v, [e16, dsp])
                vta = plsc.load_gather(vt_v, [ty16, dsp])
                plsc.store_scatter(v_v, [e16, dsp], ex16 * (va + vta))
                return c2

            lax.fori_loop(0, D, dstep, 0)
            typ_v[pl.ds(g * 16, 16)] = jax.lax.shift_right_logical(t16, 7)
        srow_v = typ_v
        pltpu.sync_copy(v_v, agg_s.at[tgt_v], add=True)
        pltpu.sync_copy(sr_v, sum_s.at[srow_v], add=True)
        for g in range(C // 16):
            e16 = lanes + (g * 16)
            col16 = jnp.bitwise_and(tgt_v[pl.ds(g * 16, 16)], 127)
            plsc.store_scatter(sr_v, [e16, col16], zero16)
        return carry

    lax.fori_loop(0, NCH, chunk, 0)
    plsc.subcore_barrier()
    pltpu.sync_copy(agg_s.at[pl.ds(row0, NR)],
                    agg_o.at[pl.ds(cid * NP + row0, NR)])
    pltpu.sync_copy(sum_s.at[pl.ds(sid * SPT, SPT)],
                    sum_o.at[pl.ds(cid * SR + sid * SPT, SPT)])


def kernel(hidden, time_emb, edge_index, edge_type, edge_emb, Wq, bq, Wk, bk,
           Wv, bv, We, be, W1, b1, W2, b2, gamma, beta):
    f32 = jnp.float32
    h2 = hidden[0]
    t2 = time_emb[0]
    src = edge_index[0]
    tgt = edge_index[1]
    pad = E2 - E
    zi = jnp.zeros((pad,), jnp.int32)
    src_p = jnp.concatenate([src, zi])
    tgt_p = jnp.concatenate([tgt, zi])
    typ_p = jnp.concatenate([edge_type, zi])

    kt2 = edge_emb @ Wk + bk
    vt2 = edge_emb @ Wv + bv
    elog = (edge_emb @ We + be).reshape(T)

    tc_proj = pl.pallas_call(
        _tc_proj_body,
        grid=(NB,),
        in_specs=[
            pl.BlockSpec((RB, D), lambda i: (i, 0)),
            pl.BlockSpec((RB, D), lambda i: (i, 0)),
            pl.BlockSpec((D, D), lambda i: (0, 0)),
            pl.BlockSpec((1, D), lambda i: (0, 0)),
            pl.BlockSpec((D, D), lambda i: (0, 0)),
            pl.BlockSpec((D, D), lambda i: (0, 0)),
        ],
        out_specs=[
            pl.BlockSpec((RB, D), lambda i: (i, 0)),
            pl.BlockSpec((RB, D), lambda i: (i, 0)),
            pl.BlockSpec((RB, D), lambda i: (i, 0)),
        ],
        out_shape=[
            jax.ShapeDtypeStruct((N, D), f32),
            jax.ShapeDtypeStruct((N, D), f32),
            jax.ShapeDtypeStruct((N, D), f32),
        ],
    )
    qn, kn, vn = tc_proj(h2, t2, Wq, bq.reshape(1, D), Wk, Wv)

    mesh = plsc.VectorSubcoreMesh(core_axis_name="c", subcore_axis_name="s")

    sc_logits = functools.partial(
        pl.kernel,
        out_type=(jax.ShapeDtypeStruct((E2,), f32),
                  jax.ShapeDtypeStruct((NW * 128,), f32)),
        mesh=mesh,
        compiler_params=pltpu.CompilerParams(needs_layout_passes=False),
        scratch_types=[
            pltpu.VMEM((C,), jnp.int32),
            pltpu.VMEM((C,), jnp.int32),
            pltpu.VMEM((C,), jnp.int32),
            pltpu.VMEM((C, D), f32),
            pltpu.VMEM((C, D), f32),
            pltpu.VMEM((T, D), f32),
            pltpu.VMEM((T,), f32),
            pltpu.VMEM((C,), f32),
            pltpu.VMEM((128,), f32),
            pltpu.SemaphoreType.DMA,
        ],
    )(_sc_logits_body)
    logit, tmax = sc_logits(qn, kn, kt2, elog, tgt_p, src_p, typ_p)

    gvec = jnp.full((16,), jnp.max(tmax), f32)
    z128 = jnp.zeros((NR, D), f32)

    sc_agg = functools.partial(
        pl.kernel,
        out_type=(jax.ShapeDtypeStruct((2 * NP, D), f32),
                  jax.ShapeDtypeStruct((2 * SR, D), f32)),
        mesh=mesh,
        compiler_params=pltpu.CompilerParams(needs_layout_passes=False),
        scratch_types=[
            pltpu.VMEM((C,), jnp.int32),
            pltpu.VMEM((C,), jnp.int32),
            pltpu.VMEM((C,), jnp.int32),
            pltpu.VMEM((C, D), f32),
            pltpu.VMEM((C, D), f32),
            pltpu.VMEM((C,), f32),
            pltpu.VMEM((T, D), f32),
            pltpu.VMEM((16,), f32),
            pltpu.VMEM_SHARED((NP, D), f32),
            pltpu.VMEM_SHARED((SR, D), f32),
            pltpu.SemaphoreType.DMA,
        ],
    )(_sc_agg_body)
    aggc, sums = sc_agg(vn, vt2, tgt_p, src_p, typ_p, logit, gvec, z128)

    a0 = aggc[:N]
    a1 = aggc[NP:NP + N]
    sm = (sums[:SR] + sums[SR:]).reshape(-1)[:N].reshape(N, 1)

    tc_fin = pl.pallas_call(
        _tc_finish_body,
        grid=(NB,),
        in_specs=[
            pl.BlockSpec((RB, D), lambda i: (i, 0)),
            pl.BlockSpec((RB, D), lambda i: (i, 0)),
            pl.BlockSpec((RB, D), lambda i: (i, 0)),
            pl.BlockSpec((RB, D), lambda i: (i, 0)),
            pl.BlockSpec((RB, 1), lambda i: (i, 0)),
            pl.BlockSpec((D, D), lambda i: (0, 0)),
            pl.BlockSpec((D, D), lambda i: (0, 0)),
            pl.BlockSpec((D, D), lambda i: (0, 0)),
            pl.BlockSpec((1, D), lambda i: (0, 0)),
            pl.BlockSpec((D, D), lambda i: (0, 0)),
            pl.BlockSpec((1, D), lambda i: (0, 0)),
            pl.BlockSpec((1, D), lambda i: (0, 0)),
            pl.BlockSpec((1, D), lambda i: (0, 0)),
        ],
        out_specs=pl.BlockSpec((RB, D), lambda i: (i, 0)),
        out_shape=jax.ShapeDtypeStruct((N, D), f32),
    )
    out = tc_fin(h2, t2, a0, a1, sm, W1[:D], W1[D:2 * D],
                 W1[2 * D:], b1.reshape(1, D), W2, b2.reshape(1, D),
                 gamma.reshape(1, D), beta.reshape(1, D))
    return out[None]


# speedup vs baseline: 2.0541x; 1.3188x over previous
import functools
import math

import jax
import jax.numpy as jnp
from jax import lax
from jax.experimental import pallas as pl
from jax.experimental.pallas import tpu as pltpu
from jax.experimental.pallas import tpu_sc as plsc

N = 10000
D = 128
E = 160000
T = 16

NC = 2
NS = 16
NW = NC * NS

C = 128
EP = ((E + NW - 1) // NW + C - 1) // C * C
E2 = EP * NW
NCH = EP // C
CB = 32
NCHB = EP // CB
NR = 632
NP = NS * NR
SPT = 80
SR = NS * SPT

SCL = 1.0 / math.sqrt(D)
RB = 2000
NB = N // RB

_HI = jax.lax.Precision.HIGHEST


def _tc_proj_body(h_ref, t_ref, wq_ref, bq_ref, wk_ref, wv_ref,
                  q_ref, k_ref, v_ref):
    h = h_ref[...]
    ht = h + t_ref[...]
    q_ref[...] = jnp.dot(ht, wq_ref[...], preferred_element_type=jnp.float32,
                         precision=_HI) + bq_ref[...]
    k_ref[...] = jnp.dot(ht, wk_ref[...], preferred_element_type=jnp.float32,
                         precision=_HI)
    v_ref[...] = jnp.dot(h, wv_ref[...], preferred_element_type=jnp.float32,
                         precision=_HI)


def _tc_finish_body(h_ref, t_ref, a0_ref, a1_ref, sn_ref, vt_ref, w1a_ref,
                    w1b_ref, w1c_ref, b1_ref, w2_ref, b2_ref, g_ref, bt_ref,
                    o_ref):
    h = h_ref[...]
    sn = sn_ref[...]
    a = (a0_ref[...] + a1_ref[...]
         + jnp.dot(sn, vt_ref[...], preferred_element_type=jnp.float32,
                   precision=_HI))
    sm = jnp.sum(sn, axis=1, keepdims=True)
    agg = jnp.where(sm > 0, a / jnp.maximum(sm, 1e-30), 0.0)
    u = (jnp.dot(h, w1a_ref[...], preferred_element_type=jnp.float32,
                 precision=_HI)
         + jnp.dot(agg, w1b_ref[...], preferred_element_type=jnp.float32,
                   precision=_HI)
         + jnp.dot(t_ref[...], w1c_ref[...],
                   preferred_element_type=jnp.float32, precision=_HI)
         + b1_ref[...])
    u = u / (1.0 + jnp.exp(-u))
    y = jnp.dot(u, w2_ref[...], preferred_element_type=jnp.float32,
                precision=_HI) + b2_ref[...] + h
    mean = jnp.mean(y, axis=1, keepdims=True)
    yc = y - mean
    var = jnp.mean(yc * yc, axis=1, keepdims=True)
    o_ref[...] = yc * jax.lax.rsqrt(var + 1e-5) * g_ref[...] + bt_ref[...]


def _sc_logits_body(qn, kn, kt2, elog, tgt, src, typ, logit_o, tmax_o,
                    tgA, tgB, srA, srB, tyA, tyB, qA, qB, kA, kB, lg_v,
                    kt_v, el_v, mx_v, semA, semB):
    cid = lax.axis_index("c")
    sid = lax.axis_index("s")
    wid = sid * NC + cid
    base = wid * EP
    lanes = lax.iota(jnp.int32, 16)

    pltpu.sync_copy(kt2, kt_v)
    pltpu.sync_copy(elog, el_v)

    def fire(ch, tg_v, sr_v, ty_v, q_v, k_v, sem):
        e0 = base + ch * C
        pltpu.sync_copy(tgt.at[pl.ds(e0, C)], tg_v)
        pltpu.sync_copy(src.at[pl.ds(e0, C)], sr_v)
        pltpu.sync_copy(typ.at[pl.ds(e0, C)], ty_v)
        pltpu.async_copy(qn.at[tg_v], q_v, sem)
        pltpu.async_copy(kn.at[sr_v], k_v, sem)

    def wait_fire(tg_v, sr_v, q_v, k_v, sem):
        pltpu.make_async_copy(qn.at[tg_v], q_v, sem).wait()
        pltpu.make_async_copy(kn.at[sr_v], k_v, sem).wait()

    def compute(ch, ty_v, q_v, k_v, mx):
        e0 = base + ch * C
        for g in range(C // 16):
            e16 = lanes + (g * 16)
            ty16 = ty_v[pl.ds(g * 16, 16)]

            def dstep(dd, acc):
                db = jnp.full((16,), dd * 8, jnp.int32)
                for j in range(8):
                    dsp = db + j
                    q16 = plsc.load_gather(q_v, [e16, dsp])
                    k16 = plsc.load_gather(k_v, [e16, dsp])
                    kt16 = plsc.load_gather(kt_v, [ty16, dsp])
                    acc = acc + q16 * (k16 + kt16)
                return acc

            acc = lax.fori_loop(0, D // 8, dstep,
                                jnp.zeros((16,), jnp.float32))
            lg = acc * SCL + plsc.load_gather(el_v, [ty16])
            eg = e0 + g * 16 + lanes
            lg = jnp.where(eg < E, lg, jnp.full((16,), -3e38, jnp.float32))
            lg_v[pl.ds(g * 16, 16)] = lg
            mx = jnp.maximum(mx, lg)
        pltpu.sync_copy(lg_v, logit_o.at[pl.ds(e0, C)])
        return mx

    fire(0, tgA, srA, tyA, qA, kA, semA)

    def loop(ii, mx):
        cha = 2 * ii
        chb = 2 * ii + 1
        fire(chb, tgB, srB, tyB, qB, kB, semB)
        wait_fire(tgA, srA, qA, kA, semA)
        mx = compute(cha, tyA, qA, kA, mx)
        fire(jnp.minimum(chb + 1, NCH - 1), tgA, srA, tyA, qA, kA, semA)
        wait_fire(tgB, srB, qB, kB, semB)
        mx = compute(chb, tyB, qB, kB, mx)
        return mx

    mx = lax.fori_loop(0, NCH // 2, loop,
                       jnp.full((16,), -3e38, jnp.float32))
    wait_fire(tgA, srA, qA, kA, semA)
    for g in range(8):
        mx_v[pl.ds(g * 16, 16)] = mx
    pltpu.sync_copy(mx_v, tmax_o.at[pl.ds(wid * 128, 128)])


def _sc_agg_body(vn, tgt, src, typ, logit, gvec, z128, agg_o, sum_o,
                 tgA, tgB, srcA, srcB, tyA, tyB, gA, gB, sA, sB, exA, exB,
                 tgSA, tgSB, rwA, rwB, csA, csB, lgA, lgB, gx_v,
                 agg_s, sum_s, semA, semB, semSA, semSB):
    cid = lax.axis_index("c")
    sid = lax.axis_index("s")
    wid = sid * NC + cid
    base = wid * EP
    row0 = sid * NR
    lanes = lax.iota(jnp.int32, 16)
    zero16 = jnp.zeros((16,), jnp.float32)
    zero16i = jnp.zeros((16,), jnp.int32)

    pltpu.sync_copy(gvec, gx_v)
    pltpu.sync_copy(z128.at[pl.ds(0, CB)], exA)
    pltpu.sync_copy(z128.at[pl.ds(0, CB)], exB)
    pltpu.sync_copy(z128.at[pl.ds(0, CB)], sA)
    pltpu.sync_copy(z128.at[pl.ds(0, CB)], sB)
    for g in range(CB // 16):
        sl = pl.ds(g * 16, 16)
        tgSA[sl] = zero16i
        tgSB[sl] = zero16i
        rwA[sl] = zero16i
        rwB[sl] = zero16i
        csA[sl] = zero16i
        csB[sl] = zero16i
    pltpu.sync_copy(z128, agg_s.at[pl.ds(row0, NR)])
    pltpu.sync_copy(z128.at[pl.ds(0, SPT)], sum_s.at[pl.ds(sid * SPT, SPT)])
    plsc.subcore_barrier()
    gx = gx_v[...]

    def fire_scatter(s_v, ex_v, tg_v, rw_v, semS):
        pltpu.async_copy(s_v, agg_s.at[tg_v], semS, add=True)
        pltpu.async_copy(ex_v, sum_s.at[rw_v], semS, add=True)

    def wait_scatter(s_v, ex_v, tg_v, rw_v, semS):
        pltpu.make_async_copy(s_v, agg_s.at[tg_v], semS).wait()
        pltpu.make_async_copy(ex_v, sum_s.at[rw_v], semS).wait()

    def fire(ch, tg_v, src_v, ty_v, g_v, lg_v, sem):
        e0 = base + ch * CB
        pltpu.sync_copy(tgt.at[pl.ds(e0, CB)], tg_v)
        pltpu.sync_copy(src.at[pl.ds(e0, CB)], src_v)
        pltpu.sync_copy(typ.at[pl.ds(e0, CB)], ty_v)
        pltpu.async_copy(vn.at[src_v], g_v, sem)
        pltpu.async_copy(logit.at[pl.ds(e0, CB)], lg_v, sem)

    def wait_fire(ch, src_v, g_v, lg_v, sem):
        e0 = base + ch * CB
        pltpu.make_async_copy(vn.at[src_v], g_v, sem).wait()
        pltpu.make_async_copy(logit.at[pl.ds(e0, CB)], lg_v, sem).wait()

    def compute(ch, tg_v, ty_v, g_v, s_v, ex_v, tgS_v, rw_v, cs_v, lg_v,
                semS, carry):
        for g in range(CB // 16):
            e16 = lanes + (g * 16)
            oldc = cs_v[pl.ds(g * 16, 16)]
            plsc.store_scatter(ex_v, [e16, oldc], zero16)
        for g in range(CB // 16):
            e16 = lanes + (g * 16)
            t16 = tg_v[pl.ds(g * 16, 16)]
            ty16 = ty_v[pl.ds(g * 16, 16)]
            ex16 = jnp.exp(lg_v[pl.ds(g * 16, 16)] - gx)
            col16 = jnp.bitwise_and(t16, 7) * 16 + ty16
            plsc.store_scatter(ex_v, [e16, col16], ex16)
            cs_v[pl.ds(g * 16, 16)] = col16
            tgS_v[pl.ds(g * 16, 16)] = t16
            rw_v[pl.ds(g * 16, 16)] = jax.lax.shift_right_logical(t16, 3)

            def dstep(dd, c2):
                db = jnp.full((16,), dd * 8, jnp.int32)
                for j in range(8):
                    dsp = db + j
                    va = plsc.load_gather(g_v, [e16, dsp])
                    plsc.store_scatter(s_v, [e16, dsp], ex16 * va)
                return c2

            lax.fori_loop(0, D // 8, dstep, 0)
        return carry

    fire_scatter(sA, exA, tgSA, rwA, semSA)
    fire_scatter(sB, exB, tgSB, rwB, semSB)
    fire(0, tgA, srcA, tyA, gA, lgA, semA)

    def loop(ii, carry):
        cha = 2 * ii
        chb = 2 * ii + 1
        wait_fire(cha, srcA, gA, lgA, semA)
        wait_scatter(sA, exA, tgSA, rwA, semSA)
        fire(chb, tgB, srcB, tyB, gB, lgB, semB)
        carry = compute(cha, tgA, tyA, gA, sA, exA, tgSA, rwA, csA, lgA,
                        semSA, carry)
        fire_scatter(sA, exA, tgSA, rwA, semSA)
        wait_fire(chb, srcB, gB, lgB, semB)
        wait_scatter(sB, exB, tgSB, rwB, semSB)
        fire(jnp.minimum(chb + 1, NCHB - 1), tgA, srcA, tyA, gA, lgA, semA)
        carry = compute(chb, tgB, tyB, gB, sB, exB, tgSB, rwB, csB, lgB,
                        semSB, carry)
        fire_scatter(sB, exB, tgSB, rwB, semSB)
        return carry

    lax.fori_loop(0, NCHB // 2, loop, 0)
    wait_fire(NCHB - 1, srcA, gA, lgA, semA)
    wait_scatter(sA, exA, tgSA, rwA, semSA)
    wait_scatter(sB, exB, tgSB, rwB, semSB)
    plsc.subcore_barrier()
    pltpu.sync_copy(agg_s.at[pl.ds(row0, NR)],
                    agg_o.at[pl.ds(cid * NP + row0, NR)])
    pltpu.sync_copy(sum_s.at[pl.ds(sid * SPT, SPT)],
                    sum_o.at[pl.ds(cid * SR + sid * SPT, SPT)])


def kernel(hidden, time_emb, edge_index, edge_type, edge_emb, Wq, bq, Wk, bk,
           Wv, bv, We, be, W1, b1, W2, b2, gamma, beta):
    f32 = jnp.float32
    h2 = hidden[0]
    t2 = time_emb[0]
    src = edge_index[0]
    tgt = edge_index[1]
    pad = E2 - E
    zi = jnp.zeros((pad,), jnp.int32)
    src_p = jnp.concatenate([src, zi])
    tgt_p = jnp.concatenate([tgt, zi])
    typ_p = jnp.concatenate([edge_type, zi])
    kt2 = edge_emb @ Wk + bk
    vt2 = edge_emb @ Wv + bv
    elog = (edge_emb @ We + be).reshape(T)

    tc_proj = pl.pallas_call(
        _tc_proj_body,
        grid=(NB,),
        in_specs=[
            pl.BlockSpec((RB, D), lambda i: (i, 0)),
            pl.BlockSpec((RB, D), lambda i: (i, 0)),
            pl.BlockSpec((D, D), lambda i: (0, 0)),
            pl.BlockSpec((1, D), lambda i: (0, 0)),
            pl.BlockSpec((D, D), lambda i: (0, 0)),
            pl.BlockSpec((D, D), lambda i: (0, 0)),
        ],
        out_specs=[
            pl.BlockSpec((RB, D), lambda i: (i, 0)),
            pl.BlockSpec((RB, D), lambda i: (i, 0)),
            pl.BlockSpec((RB, D), lambda i: (i, 0)),
        ],
        out_shape=[
            jax.ShapeDtypeStruct((N, D), f32),
            jax.ShapeDtypeStruct((N, D), f32),
            jax.ShapeDtypeStruct((N, D), f32),
        ],
    )
    qn, kn, vn = tc_proj(h2, t2, Wq, bq.reshape(1, D), Wk, Wv)

    mesh = plsc.VectorSubcoreMesh(core_axis_name="c", subcore_axis_name="s")

    sc_logits = functools.partial(
        pl.kernel,
        out_type=(jax.ShapeDtypeStruct((E2,), f32),
                  jax.ShapeDtypeStruct((NW * 128,), f32)),
        mesh=mesh,
        compiler_params=pltpu.CompilerParams(needs_layout_passes=False),
        scratch_types=[
            pltpu.VMEM((C,), jnp.int32),
            pltpu.VMEM((C,), jnp.int32),
            pltpu.VMEM((C,), jnp.int32),
            pltpu.VMEM((C,), jnp.int32),
            pltpu.VMEM((C,), jnp.int32),
            pltpu.VMEM((C,), jnp.int32),
            pltpu.VMEM((C, D), f32),
            pltpu.VMEM((C, D), f32),
            pltpu.VMEM((C, D), f32),
            pltpu.VMEM((C, D), f32),
            pltpu.VMEM((C,), f32),
            pltpu.VMEM((T, D), f32),
            pltpu.VMEM((T,), f32),
            pltpu.VMEM((128,), f32),
            pltpu.SemaphoreType.DMA,
            pltpu.SemaphoreType.DMA,
        ],
    )(_sc_logits_body)
    logit, tmax = sc_logits(qn, kn, kt2, elog, tgt_p, src_p, typ_p)

    gvec = jnp.full((16,), jnp.max(tmax), f32)
    z128 = jnp.zeros((NR, D), f32)

    sc_agg = functools.partial(
        pl.kernel,
        out_type=(jax.ShapeDtypeStruct((2 * NP, D), f32),
                  jax.ShapeDtypeStruct((2 * SR, D), f32)),
        mesh=mesh,
        compiler_params=pltpu.CompilerParams(needs_layout_passes=False),
        scratch_types=[
            pltpu.VMEM((CB,), jnp.int32),
            pltpu.VMEM((CB,), jnp.int32),
            pltpu.VMEM((CB,), jnp.int32),
            pltpu.VMEM((CB,), jnp.int32),
            pltpu.VMEM((CB,), jnp.int32),
            pltpu.VMEM((CB,), jnp.int32),
            pltpu.VMEM((CB, D), f32),
            pltpu.VMEM((CB, D), f32),
            pltpu.VMEM((CB, D), f32),
            pltpu.VMEM((CB, D), f32),
            pltpu.VMEM((CB, D), f32),
            pltpu.VMEM((CB, D), f32),
            pltpu.VMEM((CB,), jnp.int32),
            pltpu.VMEM((CB,), jnp.int32),
            pltpu.VMEM((CB,), jnp.int32),
            pltpu.VMEM((CB,), jnp.int32),
            pltpu.VMEM((CB,), jnp.int32),
            pltpu.VMEM((CB,), jnp.int32),
            pltpu.VMEM((CB,), f32),
            pltpu.VMEM((CB,), f32),
            pltpu.VMEM((16,), f32),
            pltpu.VMEM_SHARED((NP, D), f32),
            pltpu.VMEM_SHARED((SR, D), f32),
            pltpu.SemaphoreType.DMA,
            pltpu.SemaphoreType.DMA,
            pltpu.SemaphoreType.DMA,
            pltpu.SemaphoreType.DMA,
        ],
    )(_sc_agg_body)
    aggc, sums = sc_agg(vn, tgt_p, src_p, typ_p, logit, gvec, z128)

    a0 = aggc[:N]
    a1 = aggc[NP:NP + N]
    sn = (sums[:SR] + sums[SR:]).reshape(SR * 8, T)[:N]

    tc_fin = pl.pallas_call(
        _tc_finish_body,
        grid=(NB,),
        in_specs=[
            pl.BlockSpec((RB, D), lambda i: (i, 0)),
            pl.BlockSpec((RB, D), lambda i: (i, 0)),
            pl.BlockSpec((RB, D), lambda i: (i, 0)),
            pl.BlockSpec((RB, D), lambda i: (i, 0)),
            pl.BlockSpec((RB, T), lambda i: (i, 0)),
            pl.BlockSpec((T, D), lambda i: (0, 0)),
            pl.BlockSpec((D, D), lambda i: (0, 0)),
            pl.BlockSpec((D, D), lambda i: (0, 0)),
            pl.BlockSpec((D, D), lambda i: (0, 0)),
            pl.BlockSpec((1, D), lambda i: (0, 0)),
            pl.BlockSpec((D, D), lambda i: (0, 0)),
            pl.BlockSpec((1, D), lambda i: (0, 0)),
            pl.BlockSpec((1, D), lambda i: (0, 0)),
            pl.BlockSpec((1, D), lambda i: (0, 0)),
        ],
        out_specs=pl.BlockSpec((RB, D), lambda i: (i, 0)),
        out_shape=jax.ShapeDtypeStruct((N, D), f32),
    )
    out = tc_fin(h2, t2, a0, a1, sn, vt2, W1[:D], W1[D:2 * D],
                 W1[2 * D:], b1.reshape(1, D), W2, b2.reshape(1, D),
                 gamma.reshape(1, D), beta.reshape(1, D))
    return out[None]


# speedup vs baseline: 2.9084x; 1.4159x over previous
import functools
import math

import jax
import jax.numpy as jnp
from jax import lax
from jax.experimental import pallas as pl
from jax.experimental.pallas import tpu as pltpu
from jax.experimental.pallas import tpu_sc as plsc

N = 10000
D = 128
E = 160000
T = 16

NC = 2
NS = 16
NW = NC * NS

C = 128
EP = ((E + NW - 1) // NW + C - 1) // C * C
E2 = EP * NW
NCH = EP // C
CB = 32
NCHB = EP // CB
NR = 632
NP = NS * NR
SPT = 80
SR = NS * SPT

SCL = 1.0 / math.sqrt(D)
RB = 2000
NB = N // RB

_HI = jax.lax.Precision.HIGHEST


def _tc_proj_body(h_ref, t_ref, wq_ref, bq_ref, wk_ref, wv_ref, kt2t_ref,
                  el_ref, q_ref, k_ref, v_ref, lb_ref):
    h = h_ref[...]
    ht = h + t_ref[...]
    q = jnp.dot(ht, wq_ref[...], preferred_element_type=jnp.float32,
                precision=_HI) + bq_ref[...]
    q_ref[...] = q
    k_ref[...] = jnp.dot(ht, wk_ref[...], preferred_element_type=jnp.float32,
                         precision=_HI)
    v_ref[...] = jnp.dot(h, wv_ref[...], preferred_element_type=jnp.float32,
                         precision=_HI)
    lb16 = SCL * jnp.dot(q, kt2t_ref[...], preferred_element_type=jnp.float32,
                         precision=_HI) + el_ref[...]
    lb_ref[...] = jnp.concatenate(
        [lb16, jnp.zeros((RB, D - T), jnp.float32)], axis=1)


def _tc_finish_body(h_ref, t_ref, a0_ref, a1_ref, sn_ref, vt_ref, w1a_ref,
                    w1b_ref, w1c_ref, b1_ref, w2_ref, b2_ref, g_ref, bt_ref,
                    o_ref):
    h = h_ref[...]
    sn = sn_ref[...]
    a = (a0_ref[...] + a1_ref[...]
         + jnp.dot(sn, vt_ref[...], preferred_element_type=jnp.float32,
                   precision=_HI))
    sm = jnp.sum(sn, axis=1, keepdims=True)
    agg = jnp.where(sm > 0, a / jnp.maximum(sm, 1e-30), 0.0)
    u = (jnp.dot(h, w1a_ref[...], preferred_element_type=jnp.float32,
                 precision=_HI)
         + jnp.dot(agg, w1b_ref[...], preferred_element_type=jnp.float32,
                   precision=_HI)
         + jnp.dot(t_ref[...], w1c_ref[...],
                   preferred_element_type=jnp.float32, precision=_HI)
         + b1_ref[...])
    u = u / (1.0 + jnp.exp(-u))
    y = jnp.dot(u, w2_ref[...], preferred_element_type=jnp.float32,
                precision=_HI) + b2_ref[...] + h
    mean = jnp.mean(y, axis=1, keepdims=True)
    yc = y - mean
    var = jnp.mean(yc * yc, axis=1, keepdims=True)
    o_ref[...] = yc * jax.lax.rsqrt(var + 1e-5) * g_ref[...] + bt_ref[...]


def _sc_logits_body(qn, kn, lb, tgt, src, typ, logit_o, tmax_o,
                    tgA, tgB, srA, srB, tyA, tyB, qA, qB, kA, kB, lbA, lbB,
                    lg_v, mx_v, semA, semB):
    cid = lax.axis_index("c")
    sid = lax.axis_index("s")
    wid = sid * NC + cid
    base = wid * EP
    lanes = lax.iota(jnp.int32, 16)

    def fire(ch, tg_v, sr_v, ty_v, q_v, k_v, lb_v, sem):
        e0 = base + ch * C
        pltpu.sync_copy(tgt.at[pl.ds(e0, C)], tg_v)
        pltpu.sync_copy(src.at[pl.ds(e0, C)], sr_v)
        pltpu.sync_copy(typ.at[pl.ds(e0, C)], ty_v)
        pltpu.async_copy(qn.at[tg_v], q_v, sem)
        pltpu.async_copy(kn.at[sr_v], k_v, sem)
        pltpu.async_copy(lb.at[tg_v], lb_v, sem)

    def wait_fire(tg_v, sr_v, q_v, k_v, lb_v, sem):
        pltpu.make_async_copy(qn.at[tg_v], q_v, sem).wait()
        pltpu.make_async_copy(kn.at[sr_v], k_v, sem).wait()
        pltpu.make_async_copy(lb.at[tg_v], lb_v, sem).wait()

    def compute(ch, ty_v, q_v, k_v, lb_v, mx):
        e0 = base + ch * C
        for g in range(C // 16):
            ty16 = ty_v[pl.ds(g * 16, 16)]

            def estep(jj, c2):
                ebase = g * 16 + jj * 4
                for u in range(4):
                    e = ebase + u
                    acc = (q_v[e, pl.ds(0, 16)] * k_v[e, pl.ds(0, 16)])
                    for j2 in range(1, 8):
                        sl = pl.ds(j2 * 16, 16)
                        acc = acc + q_v[e, sl] * k_v[e, sl]
                    s = lax.reduce_sum(acc, axes=(0,))
                    plsc.store_scatter(
                        lg_v, [jnp.full((16,), e, jnp.int32)],
                        jnp.full((16,), s, jnp.float32), mask=lanes == 0)
                return c2

            lax.fori_loop(0, 4, estep, 0)
            e16 = lanes + (g * 16)
            raw = lg_v[pl.ds(g * 16, 16)]
            lbv = plsc.load_gather(lb_v, [e16, ty16])
            lg = raw * SCL + lbv
            eg = e0 + g * 16 + lanes
            lg = jnp.where(eg < E, lg, jnp.full((16,), -3e38, jnp.float32))
            lg_v[pl.ds(g * 16, 16)] = lg
            mx = jnp.maximum(mx, lg)
        pltpu.sync_copy(lg_v, logit_o.at[pl.ds(e0, C)])
        return mx

    fire(0, tgA, srA, tyA, qA, kA, lbA, semA)

    def loop(ii, mx):
        cha = 2 * ii
        chb = 2 * ii + 1
        fire(chb, tgB, srB, tyB, qB, kB, lbB, semB)
        wait_fire(tgA, srA, qA, kA, lbA, semA)
        mx = compute(cha, tyA, qA, kA, lbA, mx)
        fire(jnp.minimum(chb + 1, NCH - 1), tgA, srA, tyA, qA, kA, lbA,
             semA)
        wait_fire(tgB, srB, qB, kB, lbB, semB)
        mx = compute(chb, tyB, qB, kB, lbB, mx)
        return mx

    mx = lax.fori_loop(0, NCH // 2, loop,
                       jnp.full((16,), -3e38, jnp.float32))
    wait_fire(tgA, srA, qA, kA, lbA, semA)
    for g in range(8):
        mx_v[pl.ds(g * 16, 16)] = mx
    pltpu.sync_copy(mx_v, tmax_o.at[pl.ds(wid * 128, 128)])


def _sc_agg_body(vn, tgt, src, typ, logit, gvec, z128, agg_o, sum_o,
                 tgA, tgB, srcA, srcB, tyA, tyB, gA, gB, sA, sB, exA, exB,
                 tgSA, tgSB, rwA, rwB, csA, csB, lgA, lgB, ext_v, gx_v,
                 agg_s, sum_s, semA, semB, semSA, semSB):
    cid = lax.axis_index("c")
    sid = lax.axis_index("s")
    wid = sid * NC + cid
    base = wid * EP
    row0 = sid * NR
    lanes = lax.iota(jnp.int32, 16)
    zero16 = jnp.zeros((16,), jnp.float32)
    zero16i = jnp.zeros((16,), jnp.int32)

    pltpu.sync_copy(gvec, gx_v)
    pltpu.sync_copy(z128.at[pl.ds(0, CB)], exA)
    pltpu.sync_copy(z128.at[pl.ds(0, CB)], exB)
    pltpu.sync_copy(z128.at[pl.ds(0, CB)], sA)
    pltpu.sync_copy(z128.at[pl.ds(0, CB)], sB)
    for g in range(CB // 16):
        sl = pl.ds(g * 16, 16)
        tgSA[sl] = zero16i
        tgSB[sl] = zero16i
        rwA[sl] = zero16i
        rwB[sl] = zero16i
        csA[sl] = zero16i
        csB[sl] = zero16i
    pltpu.sync_copy(z128, agg_s.at[pl.ds(row0, NR)])
    pltpu.sync_copy(z128.at[pl.ds(0, SPT)], sum_s.at[pl.ds(sid * SPT, SPT)])
    plsc.subcore_barrier()
    gx = gx_v[...]

    def fire_scatter(s_v, ex_v, tg_v, rw_v, semS):
        pltpu.async_copy(s_v, agg_s.at[tg_v], semS, add=True)
        pltpu.async_copy(ex_v, sum_s.at[rw_v], semS, add=True)

    def wait_scatter(s_v, ex_v, tg_v, rw_v, semS):
        pltpu.make_async_copy(s_v, agg_s.at[tg_v], semS).wait()
        pltpu.make_async_copy(ex_v, sum_s.at[rw_v], semS).wait()

    def fire(ch, tg_v, src_v, ty_v, g_v, lg_v, sem):
        e0 = base + ch * CB
        pltpu.sync_copy(tgt.at[pl.ds(e0, CB)], tg_v)
        pltpu.sync_copy(src.at[pl.ds(e0, CB)], src_v)
        pltpu.sync_copy(typ.at[pl.ds(e0, CB)], ty_v)
        pltpu.async_copy(vn.at[src_v], g_v, sem)
        pltpu.async_copy(logit.at[pl.ds(e0, CB)], lg_v, sem)

    def wait_fire(ch, src_v, g_v, lg_v, sem):
        e0 = base + ch * CB
        pltpu.make_async_copy(vn.at[src_v], g_v, sem).wait()
        pltpu.make_async_copy(logit.at[pl.ds(e0, CB)], lg_v, sem).wait()

    def compute(ch, tg_v, ty_v, g_v, s_v, ex_v, tgS_v, rw_v, cs_v, lg_v,
                semS, carry):
        for g in range(CB // 16):
            e16 = lanes + (g * 16)
            oldc = cs_v[pl.ds(g * 16, 16)]
            plsc.store_scatter(ex_v, [e16, oldc], zero16)
        for g in range(CB // 16):
            e16 = lanes + (g * 16)
            t16 = tg_v[pl.ds(g * 16, 16)]
            ty16 = ty_v[pl.ds(g * 16, 16)]
            ex16 = jnp.exp(lg_v[pl.ds(g * 16, 16)] - gx)
            col16 = jnp.bitwise_and(t16, 7) * 16 + ty16
            plsc.store_scatter(ex_v, [e16, col16], ex16)
            cs_v[pl.ds(g * 16, 16)] = col16
            tgS_v[pl.ds(g * 16, 16)] = t16
            rw_v[pl.ds(g * 16, 16)] = jax.lax.shift_right_logical(t16, 3)
            e16g = lanes + (g * 16)

            def dstep(dd, c2):
                db = jnp.full((16,), dd * 8, jnp.int32)
                for j in range(8):
                    dsp = db + j
                    va = plsc.load_gather(g_v, [e16g, dsp])
                    plsc.store_scatter(s_v, [e16g, dsp], ex16 * va)
                return c2

            lax.fori_loop(0, D // 8, dstep, 0)
        return carry

    fire_scatter(sA, exA, tgSA, rwA, semSA)
    fire_scatter(sB, exB, tgSB, rwB, semSB)
    fire(0, tgA, srcA, tyA, gA, lgA, semA)

    def loop(ii, carry):
        cha = 2 * ii
        chb = 2 * ii + 1
        wait_fire(cha, srcA, gA, lgA, semA)
        wait_scatter(sA, exA, tgSA, rwA, semSA)
        fire(chb, tgB, srcB, tyB, gB, lgB, semB)
        carry = compute(cha, tgA, tyA, gA, sA, exA, tgSA, rwA, csA, lgA,
                        semSA, carry)
        fire_scatter(sA, exA, tgSA, rwA, semSA)
        wait_fire(chb, srcB, gB, lgB, semB)
        wait_scatter(sB, exB, tgSB, rwB, semSB)
        fire(jnp.minimum(chb + 1, NCHB - 1), tgA, srcA, tyA, gA, lgA, semA)
        carry = compute(chb, tgB, tyB, gB, sB, exB, tgSB, rwB, csB, lgB,
                        semSB, carry)
        fire_scatter(sB, exB, tgSB, rwB, semSB)
        return carry

    lax.fori_loop(0, NCHB // 2, loop, 0)
    wait_fire(NCHB - 1, srcA, gA, lgA, semA)
    wait_scatter(sA, exA, tgSA, rwA, semSA)
    wait_scatter(sB, exB, tgSB, rwB, semSB)
    plsc.subcore_barrier()
    pltpu.sync_copy(agg_s.at[pl.ds(row0, NR)],
                    agg_o.at[pl.ds(cid * NP + row0, NR)])
    pltpu.sync_copy(sum_s.at[pl.ds(sid * SPT, SPT)],
                    sum_o.at[pl.ds(cid * SR + sid * SPT, SPT)])


def kernel(hidden, time_emb, edge_index, edge_type, edge_emb, Wq, bq, Wk, bk,
           Wv, bv, We, be, W1, b1, W2, b2, gamma, beta):
    f32 = jnp.float32
    h2 = hidden[0]
    t2 = time_emb[0]
    src = edge_index[0]
    tgt = edge_index[1]
    pad = E2 - E
    zi = jnp.zeros((pad,), jnp.int32)
    src_p = jnp.concatenate([src, zi])
    tgt_p = jnp.concatenate([tgt, zi])
    typ_p = jnp.concatenate([edge_type, zi])
    kt2 = edge_emb @ Wk + bk
    vt2 = edge_emb @ Wv + bv
    elog = (edge_emb @ We + be).reshape(T)

    tc_proj = pl.pallas_call(
        _tc_proj_body,
        grid=(NB,),
        in_specs=[
            pl.BlockSpec((RB, D), lambda i: (i, 0)),
            pl.BlockSpec((RB, D), lambda i: (i, 0)),
            pl.BlockSpec((D, D), lambda i: (0, 0)),
            pl.BlockSpec((1, D), lambda i: (0, 0)),
            pl.BlockSpec((D, D), lambda i: (0, 0)),
            pl.BlockSpec((D, D), lambda i: (0, 0)),
            pl.BlockSpec((D, T), lambda i: (0, 0)),
            pl.BlockSpec((1, T), lambda i: (0, 0)),
        ],
        out_specs=[
            pl.BlockSpec((RB, D), lambda i: (i, 0)),
            pl.BlockSpec((RB, D), lambda i: (i, 0)),
            pl.BlockSpec((RB, D), lambda i: (i, 0)),
            pl.BlockSpec((RB, D), lambda i: (i, 0)),
        ],
        out_shape=[
            jax.ShapeDtypeStruct((N, D), f32),
            jax.ShapeDtypeStruct((N, D), f32),
            jax.ShapeDtypeStruct((N, D), f32),
            jax.ShapeDtypeStruct((N, D), f32),
        ],
    )
    qn, kn, vn, lb128 = tc_proj(h2, t2, Wq, bq.reshape(1, D), Wk, Wv,
                                kt2.T, elog.reshape(1, T))

    mesh = plsc.VectorSubcoreMesh(core_axis_name="c", subcore_axis_name="s")

    sc_logits = functools.partial(
        pl.kernel,
        out_type=(jax.ShapeDtypeStruct((E2,), f32),
                  jax.ShapeDtypeStruct((NW * 128,), f32)),
        mesh=mesh,
        compiler_params=pltpu.CompilerParams(needs_layout_passes=False),
        scratch_types=[
            pltpu.VMEM((C,), jnp.int32),
            pltpu.VMEM((C,), jnp.int32),
            pltpu.VMEM((C,), jnp.int32),
            pltpu.VMEM((C,), jnp.int32),
            pltpu.VMEM((C,), jnp.int32),
            pltpu.VMEM((C,), jnp.int32),
            pltpu.VMEM((C, D), f32),
            pltpu.VMEM((C, D), f32),
            pltpu.VMEM((C, D), f32),
            pltpu.VMEM((C, D), f32),
            pltpu.VMEM((C, D), f32),
            pltpu.VMEM((C, D), f32),
            pltpu.VMEM((C,), f32),
            pltpu.VMEM((128,), f32),
            pltpu.SemaphoreType.DMA,
            pltpu.SemaphoreType.DMA,
        ],
    )(_sc_logits_body)
    logit, tmax = sc_logits(qn, kn, lb128, tgt_p, src_p, typ_p)

    gvec = jnp.full((16,), jnp.max(tmax), f32)
    z128 = jnp.zeros((NR, D), f32)

    sc_agg = functools.partial(
        pl.kernel,
        out_type=(jax.ShapeDtypeStruct((2 * NP, D), f32),
                  jax.ShapeDtypeStruct((2 * SR, D), f32)),
        mesh=mesh,
        compiler_params=pltpu.CompilerParams(needs_layout_passes=False),
        scratch_types=[
            pltpu.VMEM((CB,), jnp.int32),
            pltpu.VMEM((CB,), jnp.int32),
            pltpu.VMEM((CB,), jnp.int32),
            pltpu.VMEM((CB,), jnp.int32),
            pltpu.VMEM((CB,), jnp.int32),
            pltpu.VMEM((CB,), jnp.int32),
            pltpu.VMEM((CB, D), f32),
            pltpu.VMEM((CB, D), f32),
            pltpu.VMEM((CB, D), f32),
            pltpu.VMEM((CB, D), f32),
            pltpu.VMEM((CB, D), f32),
            pltpu.VMEM((CB, D), f32),
            pltpu.VMEM((CB,), jnp.int32),
            pltpu.VMEM((CB,), jnp.int32),
            pltpu.VMEM((CB,), jnp.int32),
            pltpu.VMEM((CB,), jnp.int32),
            pltpu.VMEM((CB,), jnp.int32),
            pltpu.VMEM((CB,), jnp.int32),
            pltpu.VMEM((CB,), f32),
            pltpu.VMEM((CB,), f32),
            pltpu.VMEM((16,), f32),
            pltpu.VMEM((16,), f32),
            pltpu.VMEM_SHARED((NP, D), f32),
            pltpu.VMEM_SHARED((SR, D), f32),
            pltpu.SemaphoreType.DMA,
            pltpu.SemaphoreType.DMA,
            pltpu.SemaphoreType.DMA,
            pltpu.SemaphoreType.DMA,
        ],
    )(_sc_agg_body)
    aggc, sums = sc_agg(vn, tgt_p, src_p, typ_p, logit, gvec, z128)

    a0 = aggc[:N]
    a1 = aggc[NP:NP + N]
    sn = (sums[:SR] + sums[SR:]).reshape(SR * 8, T)[:N]

    tc_fin = pl.pallas_call(
        _tc_finish_body,
        grid=(NB,),
        in_specs=[
            pl.BlockSpec((RB, D), lambda i: (i, 0)),
            pl.BlockSpec((RB, D), lambda i: (i, 0)),
            pl.BlockSpec((RB, D), lambda i: (i, 0)),
            pl.BlockSpec((RB, D), lambda i: (i, 0)),
            pl.BlockSpec((RB, T), lambda i: (i, 0)),
            pl.BlockSpec((T, D), lambda i: (0, 0)),
            pl.BlockSpec((D, D), lambda i: (0, 0)),
            pl.BlockSpec((D, D), lambda i: (0, 0)),
            pl.BlockSpec((D, D), lambda i: (0, 0)),
            pl.BlockSpec((1, D), lambda i: (0, 0)),
            pl.BlockSpec((D, D), lambda i: (0, 0)),
            pl.BlockSpec((1, D), lambda i: (0, 0)),
            pl.BlockSpec((1, D), lambda i: (0, 0)),
            pl.BlockSpec((1, D), lambda i: (0, 0)),
        ],
        out_specs=pl.BlockSpec((RB, D), lambda i: (i, 0)),
        out_shape=jax.ShapeDtypeStruct((N, D), f32),
    )
    out = tc_fin(h2, t2, a0, a1, sn, vt2, W1[:D], W1[D:2 * D],
                 W1[2 * D:], b1.reshape(1, D), W2, b2.reshape(1, D),
                 gamma.reshape(1, D), beta.reshape(1, D))
    return out[None]


# speedup vs baseline: 4.1808x; 1.4375x over previous
import functools
import math

import jax
import jax.numpy as jnp
from jax import lax
from jax.experimental import pallas as pl
from jax.experimental.pallas import tpu as pltpu
from jax.experimental.pallas import tpu_sc as plsc

N = 10000
D = 128
E = 160000
T = 16

NC = 2
NS = 16
NW = NC * NS

C = 128
EP = ((E + NW - 1) // NW + C - 1) // C * C
E2 = EP * NW
NCH = EP // C
CB = 32
NCHB = EP // CB
NR = 632
NP = NS * NR
SPT = 80
SR = NS * SPT

SCL = 1.0 / math.sqrt(D)
RB = 2000
NB = N // RB

_HI = jax.lax.Precision.HIGHEST


def _tc_proj_body(h_ref, t_ref, wq_ref, bq_ref, wk_ref, wv_ref, kt2t_ref,
                  el_ref, q_ref, k_ref, v_ref, lb_ref):
    h = h_ref[...]
    ht = h + t_ref[...]
    q = jnp.dot(ht, wq_ref[...], preferred_element_type=jnp.float32,
                precision=_HI) + bq_ref[...]
    q_ref[...] = q
    k_ref[...] = jnp.dot(ht, wk_ref[...], preferred_element_type=jnp.float32,
                         precision=_HI)
    v_ref[...] = jnp.dot(h, wv_ref[...], preferred_element_type=jnp.float32,
                         precision=_HI)
    lb16 = SCL * jnp.dot(q, kt2t_ref[...], preferred_element_type=jnp.float32,
                         precision=_HI) + el_ref[...]
    lb_ref[...] = jnp.concatenate(
        [lb16, jnp.zeros((RB, D - T), jnp.float32)], axis=1)


def _tc_finish_body(h_ref, t_ref, a0_ref, a1_ref, sn_ref, vt_ref, w1a_ref,
                    w1b_ref, w1c_ref, b1_ref, w2_ref, b2_ref, g_ref, bt_ref,
                    o_ref):
    h = h_ref[...]
    sn = sn_ref[...]
    a = (a0_ref[...] + a1_ref[...]
         + jnp.dot(sn, vt_ref[...], preferred_element_type=jnp.float32,
                   precision=_HI))
    sm = jnp.sum(sn, axis=1, keepdims=True)
    agg = jnp.where(sm > 0, a / jnp.maximum(sm, 1e-30), 0.0)
    u = (jnp.dot(h, w1a_ref[...], preferred_element_type=jnp.float32,
                 precision=_HI)
         + jnp.dot(agg, w1b_ref[...], preferred_element_type=jnp.float32,
                   precision=_HI)
         + jnp.dot(t_ref[...], w1c_ref[...],
                   preferred_element_type=jnp.float32, precision=_HI)
         + b1_ref[...])
    u = u / (1.0 + jnp.exp(-u))
    y = jnp.dot(u, w2_ref[...], preferred_element_type=jnp.float32,
                precision=_HI) + b2_ref[...] + h
    mean = jnp.mean(y, axis=1, keepdims=True)
    yc = y - mean
    var = jnp.mean(yc * yc, axis=1, keepdims=True)
    o_ref[...] = yc * jax.lax.rsqrt(var + 1e-5) * g_ref[...] + bt_ref[...]


def _sc_logits_body(qn, kn, lb, tgt, src, typ, logit_o, tmax_o,
                    tgA, tgB, srA, srB, tyA, tyB, qA, qB, kA, kB, lbA, lbB,
                    lg_v, mx_v, semA, semB):
    cid = lax.axis_index("c")
    sid = lax.axis_index("s")
    wid = sid * NC + cid
    base = wid * EP
    lanes = lax.iota(jnp.int32, 16)

    def fire(ch, tg_v, sr_v, ty_v, q_v, k_v, lb_v, sem):
        e0 = base + ch * C
        pltpu.sync_copy(tgt.at[pl.ds(e0, C)], tg_v)
        pltpu.sync_copy(src.at[pl.ds(e0, C)], sr_v)
        pltpu.sync_copy(typ.at[pl.ds(e0, C)], ty_v)
        pltpu.async_copy(qn.at[tg_v], q_v, sem)
        pltpu.async_copy(kn.at[sr_v], k_v, sem)
        pltpu.async_copy(lb.at[tg_v], lb_v, sem)

    def wait_fire(tg_v, sr_v, q_v, k_v, lb_v, sem):
        pltpu.make_async_copy(qn.at[tg_v], q_v, sem).wait()
        pltpu.make_async_copy(kn.at[sr_v], k_v, sem).wait()
        pltpu.make_async_copy(lb.at[tg_v], lb_v, sem).wait()

    def compute(ch, ty_v, q_v, k_v, lb_v, mx):
        e0 = base + ch * C
        for g in range(C // 16):
            ty16 = ty_v[pl.ds(g * 16, 16)]

            def estep(jj, c2):
                ebase = g * 16 + jj * 4
                for u in range(4):
                    e = ebase + u
                    acc = (q_v[e, pl.ds(0, 16)] * k_v[e, pl.ds(0, 16)])
                    for j2 in range(1, 8):
                        sl = pl.ds(j2 * 16, 16)
                        acc = acc + q_v[e, sl] * k_v[e, sl]
                    s = lax.reduce_sum(acc, axes=(0,))
                    plsc.store_scatter(
                        lg_v, [jnp.full((16,), e, jnp.int32)],
                        jnp.full((16,), s, jnp.float32), mask=lanes == 0)
                return c2

            lax.fori_loop(0, 4, estep, 0)
            e16 = lanes + (g * 16)
            raw = lg_v[pl.ds(g * 16, 16)]
            lbv = plsc.load_gather(lb_v, [e16, ty16])
            lg = raw * SCL + lbv
            eg = e0 + g * 16 + lanes
            lg = jnp.where(eg < E, lg, jnp.full((16,), -3e38, jnp.float32))
            lg_v[pl.ds(g * 16, 16)] = lg
            mx = jnp.maximum(mx, lg)
        pltpu.sync_copy(lg_v, logit_o.at[pl.ds(e0, C)])
        return mx

    fire(0, tgA, srA, tyA, qA, kA, lbA, semA)

    def loop(ii, mx):
        cha = 2 * ii
        chb = 2 * ii + 1
        fire(chb, tgB, srB, tyB, qB, kB, lbB, semB)
        wait_fire(tgA, srA, qA, kA, lbA, semA)
        mx = compute(cha, tyA, qA, kA, lbA, mx)
        fire(jnp.minimum(chb + 1, NCH - 1), tgA, srA, tyA, qA, kA, lbA,
             semA)
        wait_fire(tgB, srB, qB, kB, lbB, semB)
        mx = compute(chb, tyB, qB, kB, lbB, mx)
        return mx

    mx = lax.fori_loop(0, NCH // 2, loop,
                       jnp.full((16,), -3e38, jnp.float32))
    wait_fire(tgA, srA, qA, kA, lbA, semA)
    for g in range(8):
        mx_v[pl.ds(g * 16, 16)] = mx
    pltpu.sync_copy(mx_v, tmax_o.at[pl.ds(wid * 128, 128)])


def _sc_agg_body(vn, tgt, src, typ, logit, gvec, z128, agg_o, sum_o,
                 tgA, tgB, srcA, srcB, tyA, tyB, gA, gB, sA, sB, exA, exB,
                 tgSA, tgSB, rwA, rwB, csA, csB, lgA, lgB, ext_v, gx_v,
                 agg_s, sum_s, semA, semB, semSA, semSB):
    cid = lax.axis_index("c")
    sid = lax.axis_index("s")
    wid = sid * NC + cid
    base = wid * EP
    row0 = sid * NR
    lanes = lax.iota(jnp.int32, 16)
    zero16 = jnp.zeros((16,), jnp.float32)
    zero16i = jnp.zeros((16,), jnp.int32)

    pltpu.sync_copy(gvec, gx_v)
    pltpu.sync_copy(z128.at[pl.ds(0, CB)], exA)
    pltpu.sync_copy(z128.at[pl.ds(0, CB)], exB)
    pltpu.sync_copy(z128.at[pl.ds(0, CB)], sA)
    pltpu.sync_copy(z128.at[pl.ds(0, CB)], sB)
    for g in range(CB // 16):
        sl = pl.ds(g * 16, 16)
        tgSA[sl] = zero16i
        tgSB[sl] = zero16i
        rwA[sl] = zero16i
        rwB[sl] = zero16i
        csA[sl] = zero16i
        csB[sl] = zero16i
    pltpu.sync_copy(z128, agg_s.at[pl.ds(row0, NR)])
    pltpu.sync_copy(z128.at[pl.ds(0, SPT)], sum_s.at[pl.ds(sid * SPT, SPT)])
    plsc.subcore_barrier()
    gx = gx_v[...]

    def fire_scatter(s_v, ex_v, tg_v, rw_v, semS):
        pltpu.async_copy(s_v, agg_s.at[tg_v], semS, add=True)
        pltpu.async_copy(ex_v, sum_s.at[rw_v], semS, add=True)

    def wait_scatter(s_v, ex_v, tg_v, rw_v, semS):
        pltpu.make_async_copy(s_v, agg_s.at[tg_v], semS).wait()
        pltpu.make_async_copy(ex_v, sum_s.at[rw_v], semS).wait()

    def fire(ch, tg_v, src_v, ty_v, g_v, lg_v, sem):
        e0 = base + ch * CB
        pltpu.sync_copy(tgt.at[pl.ds(e0, CB)], tg_v)
        pltpu.sync_copy(src.at[pl.ds(e0, CB)], src_v)
        pltpu.sync_copy(typ.at[pl.ds(e0, CB)], ty_v)
        pltpu.async_copy(vn.at[src_v], g_v, sem)
        pltpu.async_copy(logit.at[pl.ds(e0, CB)], lg_v, sem)

    def wait_fire(ch, src_v, g_v, lg_v, sem):
        e0 = base + ch * CB
        pltpu.make_async_copy(vn.at[src_v], g_v, sem).wait()
        pltpu.make_async_copy(logit.at[pl.ds(e0, CB)], lg_v, sem).wait()

    def compute(ch, tg_v, ty_v, g_v, s_v, ex_v, tgS_v, rw_v, cs_v, lg_v,
                semS, carry):
        for g in range(CB // 16):
            e16 = lanes + (g * 16)
            oldc = cs_v[pl.ds(g * 16, 16)]
            plsc.store_scatter(ex_v, [e16, oldc], zero16)
        for g in range(CB // 16):
            e16 = lanes + (g * 16)
            t16 = tg_v[pl.ds(g * 16, 16)]
            ty16 = ty_v[pl.ds(g * 16, 16)]
            ex16 = jnp.exp(lg_v[pl.ds(g * 16, 16)] - gx)
            col16 = jnp.bitwise_and(t16, 7) * 16 + ty16
            plsc.store_scatter(ex_v, [e16, col16], ex16)
            cs_v[pl.ds(g * 16, 16)] = col16
            tgS_v[pl.ds(g * 16, 16)] = t16
            rw_v[pl.ds(g * 16, 16)] = jax.lax.shift_right_logical(t16, 3)
            for j in range(16):
                e = g * 16 + j
                esplat = jnp.full((16,), e, jnp.int32)
                exs = lax.reduce_sum(jnp.where(lanes == j, ex16, 0.0),
                                     axes=(0,))
                exE = jnp.full((16,), exs, jnp.float32)
                for j2 in range(8):
                    d16 = lanes + (j2 * 16)
                    va = plsc.load_gather(g_v, [esplat, d16])
                    plsc.store_scatter(s_v, [esplat, d16], exE * va)
        return carry

    fire_scatter(sA, exA, tgSA, rwA, semSA)
    fire_scatter(sB, exB, tgSB, rwB, semSB)
    fire(0, tgA, srcA, tyA, gA, lgA, semA)

    def loop(ii, carry):
        cha = 2 * ii
        chb = 2 * ii + 1
        wait_fire(cha, srcA, gA, lgA, semA)
        wait_scatter(sA, exA, tgSA, rwA, semSA)
        fire(chb, tgB, srcB, tyB, gB, lgB, semB)
        carry = compute(cha, tgA, tyA, gA, sA, exA, tgSA, rwA, csA, lgA,
                        semSA, carry)
        fire_scatter(sA, exA, tgSA, rwA, semSA)
        wait_fire(chb, srcB, gB, lgB, semB)
        wait_scatter(sB, exB, tgSB, rwB, semSB)
        fire(jnp.minimum(chb + 1, NCHB - 1), tgA, srcA, tyA, gA, lgA, semA)
        carry = compute(chb, tgB, tyB, gB, sB, exB, tgSB, rwB, csB, lgB,
                        semSB, carry)
        fire_scatter(sB, exB, tgSB, rwB, semSB)
        return carry

    lax.fori_loop(0, NCHB // 2, loop, 0)
    wait_fire(NCHB - 1, srcA, gA, lgA, semA)
    wait_scatter(sA, exA, tgSA, rwA, semSA)
    wait_scatter(sB, exB, tgSB, rwB, semSB)
    plsc.subcore_barrier()
    pltpu.sync_copy(agg_s.at[pl.ds(row0, NR)],
                    agg_o.at[pl.ds(cid * NP + row0, NR)])
    pltpu.sync_copy(sum_s.at[pl.ds(sid * SPT, SPT)],
                    sum_o.at[pl.ds(cid * SR + sid * SPT, SPT)])


def kernel(hidden, time_emb, edge_index, edge_type, edge_emb, Wq, bq, Wk, bk,
           Wv, bv, We, be, W1, b1, W2, b2, gamma, beta):
    f32 = jnp.float32
    h2 = hidden[0]
    t2 = time_emb[0]
    src = edge_index[0]
    tgt = edge_index[1]
    pad = E2 - E
    zi = jnp.zeros((pad,), jnp.int32)
    src_p = jnp.concatenate([src, zi])
    tgt_p = jnp.concatenate([tgt, zi])
    typ_p = jnp.concatenate([edge_type, zi])
    kt2 = edge_emb @ Wk + bk
    vt2 = edge_emb @ Wv + bv
    elog = (edge_emb @ We + be).reshape(T)

    tc_proj = pl.pallas_call(
        _tc_proj_body,
        grid=(NB,),
        in_specs=[
            pl.BlockSpec((RB, D), lambda i: (i, 0)),
            pl.BlockSpec((RB, D), lambda i: (i, 0)),
            pl.BlockSpec((D, D), lambda i: (0, 0)),
            pl.BlockSpec((1, D), lambda i: (0, 0)),
            pl.BlockSpec((D, D), lambda i: (0, 0)),
            pl.BlockSpec((D, D), lambda i: (0, 0)),
            pl.BlockSpec((D, T), lambda i: (0, 0)),
            pl.BlockSpec((1, T), lambda i: (0, 0)),
        ],
        out_specs=[
            pl.BlockSpec((RB, D), lambda i: (i, 0)),
            pl.BlockSpec((RB, D), lambda i: (i, 0)),
            pl.BlockSpec((RB, D), lambda i: (i, 0)),
            pl.BlockSpec((RB, D), lambda i: (i, 0)),
        ],
        out_shape=[
            jax.ShapeDtypeStruct((N, D), f32),
            jax.ShapeDtypeStruct((N, D), f32),
            jax.ShapeDtypeStruct((N, D), f32),
            jax.ShapeDtypeStruct((N, D), f32),
        ],
    )
    qn, kn, vn, lb128 = tc_proj(h2, t2, Wq, bq.reshape(1, D), Wk, Wv,
                                kt2.T, elog.reshape(1, T))

    mesh = plsc.VectorSubcoreMesh(core_axis_name="c", subcore_axis_name="s")

    sc_logits = functools.partial(
        pl.kernel,
        out_type=(jax.ShapeDtypeStruct((E2,), f32),
                  jax.ShapeDtypeStruct((NW * 128,), f32)),
        mesh=mesh,
        compiler_params=pltpu.CompilerParams(needs_layout_passes=False),
        scratch_types=[
            pltpu.VMEM((C,), jnp.int32),
            pltpu.VMEM((C,), jnp.int32),
            pltpu.VMEM((C,), jnp.int32),
            pltpu.VMEM((C,), jnp.int32),
            pltpu.VMEM((C,), jnp.int32),
            pltpu.VMEM((C,), jnp.int32),
            pltpu.VMEM((C, D), f32),
            pltpu.VMEM((C, D), f32),
            pltpu.VMEM((C, D), f32),
            pltpu.VMEM((C, D), f32),
            pltpu.VMEM((C, D), f32),
            pltpu.VMEM((C, D), f32),
            pltpu.VMEM((C,), f32),
            pltpu.VMEM((128,), f32),
            pltpu.SemaphoreType.DMA,
            pltpu.SemaphoreType.DMA,
        ],
    )(_sc_logits_body)
    logit, tmax = sc_logits(qn, kn, lb128, tgt_p, src_p, typ_p)

    gvec = jnp.full((16,), jnp.max(tmax), f32)
    z128 = jnp.zeros((NR, D), f32)

    sc_agg = functools.partial(
        pl.kernel,
        out_type=(jax.ShapeDtypeStruct((2 * NP, D), f32),
                  jax.ShapeDtypeStruct((2 * SR, D), f32)),
        mesh=mesh,
        compiler_params=pltpu.CompilerParams(needs_layout_passes=False),
        scratch_types=[
            pltpu.VMEM((CB,), jnp.int32),
            pltpu.VMEM((CB,), jnp.int32),
            pltpu.VMEM((CB,), jnp.int32),
            pltpu.VMEM((CB,), jnp.int32),
            pltpu.VMEM((CB,), jnp.int32),
            pltpu.VMEM((CB,), jnp.int32),
            pltpu.VMEM((CB, D), f32),
            pltpu.VMEM((CB, D), f32),
            pltpu.VMEM((CB, D), f32),
            pltpu.VMEM((CB, D), f32),
            pltpu.VMEM((CB, D), f32),
            pltpu.VMEM((CB, D), f32),
            pltpu.VMEM((CB,), jnp.int32),
            pltpu.VMEM((CB,), jnp.int32),
            pltpu.VMEM((CB,), jnp.int32),
            pltpu.VMEM((CB,), jnp.int32),
            pltpu.VMEM((CB,), jnp.int32),
            pltpu.VMEM((CB,), jnp.int32),
            pltpu.VMEM((CB,), f32),
            pltpu.VMEM((CB,), f32),
            pltpu.VMEM((16,), f32),
            pltpu.VMEM((16,), f32),
            pltpu.VMEM_SHARED((NP, D), f32),
            pltpu.VMEM_SHARED((SR, D), f32),
            pltpu.SemaphoreType.DMA,
            pltpu.SemaphoreType.DMA,
            pltpu.SemaphoreType.DMA,
            pltpu.SemaphoreType.DMA,
        ],
    )(_sc_agg_body)
    aggc, sums = sc_agg(vn, tgt_p, src_p, typ_p, logit, gvec, z128)

    a0 = aggc[:N]
    a1 = aggc[NP:NP + N]
    sn = (sums[:SR] + sums[SR:]).reshape(SR * 8, T)[:N]

    tc_fin = pl.pallas_call(
        _tc_finish_body,
        grid=(NB,),
        in_specs=[
            pl.BlockSpec((RB, D), lambda i: (i, 0)),
            pl.BlockSpec((RB, D), lambda i: (i, 0)),
            pl.BlockSpec((RB, D), lambda i: (i, 0)),
            pl.BlockSpec((RB, D), lambda i: (i, 0)),
            pl.BlockSpec((RB, T), lambda i: (i, 0)),
            pl.BlockSpec((T, D), lambda i: (0, 0)),
            pl.BlockSpec((D, D), lambda i: (0, 0)),
            pl.BlockSpec((D, D), lambda i: (0, 0)),
            pl.BlockSpec((D, D), lambda i: (0, 0)),
            pl.BlockSpec((1, D), lambda i: (0, 0)),
            pl.BlockSpec((D, D), lambda i: (0, 0)),
            pl.BlockSpec((1, D), lambda i: (0, 0)),
            pl.BlockSpec((1, D), lambda i: (0, 0)),
            pl.BlockSpec((1, D), lambda i: (0, 0)),
        ],
        out_specs=pl.BlockSpec((RB, D), lambda i: (i, 0)),
        out_shape=jax.ShapeDtypeStruct((N, D), f32),
    )
    out = tc_fin(h2, t2, a0, a1, sn, vt2, W1[:D], W1[D:2 * D],
                 W1[2 * D:], b1.reshape(1, D), W2, b2.reshape(1, D),
                 gamma.reshape(1, D), beta.reshape(1, D))
    return out[None]
